# baseline (speedup 1.0000x reference)
import functools

import jax
import jax.numpy as jnp
from jax import lax
from jax.experimental import pallas as pl
from jax.experimental.pallas import tpu as pltpu
from jax.experimental.pallas import tpu_sc as plsc


_NC = 2
_NS = 16
_NW = _NC * _NS
_LANES = 16

_CB = 8
_SPLITS = (104, 96)


def _sum_block(rows_ref, out_ref, out_row0, n_rows, l_per_row):
    unroll = 8
    steps = l_per_row // unroll

    for i in range(n_rows):
        flat0 = i * l_per_row

        def body(t, accs):
            accs = list(accs)
            r = flat0 + t * unroll
            for u in range(unroll):
                accs[2 * (u % 4)] = (
                    accs[2 * (u % 4)] + rows_ref[r + u, 0:16])
                accs[2 * (u % 4) + 1] = (
                    accs[2 * (u % 4) + 1] + rows_ref[r + u, 16:32])
            return tuple(accs)

        z = jnp.zeros((_LANES,), jnp.float32)
        accs = lax.fori_loop(0, steps, body, (z,) * 8)
        out_ref[out_row0 + i, 0:16] = (accs[0] + accs[2]) + (accs[4] + accs[6])
        out_ref[out_row0 + i, 16:32] = (accs[1] + accs[3]) + (accs[5] + accs[7])


def _make_sc_embed_sum(B, V, T, L):
    assert T == 32 and sum(_SPLITS) == L
    b_per_w = B // _NW
    n_blocks = b_per_w // _CB
    rows_per_block = _CB * L
    mesh = plsc.VectorSubcoreMesh(core_axis_name="c", subcore_axis_name="s")

    @functools.partial(
        pl.kernel,
        out_type=jax.ShapeDtypeStruct((B, T), jnp.float32),
        mesh=mesh,
        compiler_params=pltpu.CompilerParams(use_tc_tiling_on_sc=False),
        scratch_types=[
            pltpu.VMEM((2, _CB, L), jnp.int32),
            pltpu.VMEM((rows_per_block, T), jnp.float32),
            pltpu.VMEM((rows_per_block, T), jnp.float32),
            pltpu.VMEM((b_per_w, T), jnp.float32),
            pltpu.SemaphoreType.DMA,
            pltpu.SemaphoreType.DMA,
            pltpu.SemaphoreType.DMA,
            pltpu.SemaphoreType.DMA,
        ],
    )
    def sc_embed_sum(x_hbm, tab_hbm, out_hbm, idx_v, rows0, rows1, out_v,
                     gsem0, gsem1, isem0, isem1):
        wid = lax.axis_index("s") * _NC + lax.axis_index("c")
        base = wid * b_per_w
        rows_bufs = (rows0, rows1)
        gsems = (gsem0, gsem1)
        isems = (isem0, isem1)

        def idx_src(kb):
            return x_hbm.at[pl.ds(base + kb * _CB, _CB)]

        def fire_gathers(kb_buf, rows_ref, sem):
            for i in range(_CB):
                off = 0
                for g in _SPLITS:
                    pltpu.async_copy(
                        tab_hbm.at[idx_v.at[kb_buf, i, pl.ds(off, g)]],
                        rows_ref.at[pl.ds(i * L + off, g)],
                        sem,
                    )
                    off += g

        def drain_gathers(rows_ref, sem):
            pltpu.make_async_copy(
                tab_hbm.at[pl.ds(0, rows_per_block)], rows_ref, sem
            ).wait()

        pltpu.sync_copy(idx_src(0), idx_v.at[0])
        fire_gathers(0, rows0, gsem0)
        pltpu.async_copy(idx_src(1), idx_v.at[1], isem1)

        def half_step(kb, cur):
            rows_c = rows_bufs[cur]
            rows_n = rows_bufs[1 - cur]
            drain_gathers(rows_c, gsems[cur])

            @pl.when(kb + 2 < n_blocks)
            def _():
                pltpu.async_copy(idx_src(kb + 2), idx_v.at[cur], isems[cur])

            @pl.when(kb + 1 < n_blocks)
            def _():
                pltpu.make_async_copy(
                    idx_src(kb + 1), idx_v.at[1 - cur], isems[1 - cur]
                ).wait()
                fire_gathers(1 - cur, rows_n, gsems[1 - cur])

            _sum_block(rows_c, out_v, kb * _CB, _CB, L)

        def body(t, carry):
            half_step(2 * t, 0)
            half_step(2 * t + 1, 1)
            return carry

        lax.fori_loop(0, n_blocks // 2, body, 0)
        pltpu.sync_copy(out_v, out_hbm.at[pl.ds(base, b_per_w)])

    return sc_embed_sum


def _logsoftmax_body(s_ref, b_ref, o_ref):
    s = s_ref[...] + b_ref[...]
    m = jnp.max(s, axis=-1, keepdims=True)
    e = jnp.exp(s - m)
    lse = jnp.log(jnp.sum(e, axis=-1, keepdims=True))
    o_ref[...] = (s - m) - lse


def _tc_log_softmax(scores, bias):
    B, T = scores.shape
    blk = min(2048, B)
    return pl.pallas_call(
        _logsoftmax_body,
        out_shape=jax.ShapeDtypeStruct((B, T), jnp.float32),
        grid=(B // blk,),
        in_specs=[
            pl.BlockSpec((blk, T), lambda i: (i, 0)),
            pl.BlockSpec((1, T), lambda i: (0, 0)),
        ],
        out_specs=pl.BlockSpec((blk, T), lambda i: (i, 0)),
    )(scores, bias.reshape(1, T))


def kernel(x, embed_weight, bow_bias):
    B, L = x.shape
    V, T = embed_weight.shape
    scores = _make_sc_embed_sum(B, V, T, L)(x, embed_weight)
    return _tc_log_softmax(scores, bow_bias)


# speedup vs baseline: 1.0309x; 1.0309x over previous
import functools

import jax
import jax.numpy as jnp
from jax import lax
from jax.experimental import pallas as pl
from jax.experimental.pallas import tpu as pltpu
from jax.experimental.pallas import tpu_sc as plsc


_NC = 2
_NS = 16
_NW = _NC * _NS
_LANES = 16

_CB = 8
_SPLITS = (104, 96)


def _sum_block(rows_ref, out_ref, out_row0, n_rows, l_per_row):
    unroll = 8
    steps = l_per_row // unroll

    for i in range(n_rows):
        flat0 = i * l_per_row

        def body(t, accs):
            accs = list(accs)
            r = flat0 + t * unroll
            for u in range(unroll):
                accs[2 * (u % 4)] = (
                    accs[2 * (u % 4)] + rows_ref[r + u, 0:16])
                accs[2 * (u % 4) + 1] = (
                    accs[2 * (u % 4) + 1] + rows_ref[r + u, 16:32])
            return tuple(accs)

        z = jnp.zeros((_LANES,), jnp.float32)
        accs = lax.fori_loop(0, steps, body, (z,) * 8)
        out_ref[out_row0 + i, 0:16] = (accs[0] + accs[2]) + (accs[4] + accs[6])
        out_ref[out_row0 + i, 16:32] = (accs[1] + accs[3]) + (accs[5] + accs[7])


def _make_sc_embed_sum(B, V, T, L):
    assert T == 32 and sum(_SPLITS) == L
    b_per_w = B // _NW
    n_blocks = b_per_w // _CB
    rows_per_block = _CB * L
    mesh = plsc.VectorSubcoreMesh(core_axis_name="c", subcore_axis_name="s")

    @functools.partial(
        pl.kernel,
        out_type=jax.ShapeDtypeStruct((B, T), jnp.float32),
        mesh=mesh,
        compiler_params=pltpu.CompilerParams(use_tc_tiling_on_sc=False),
        scratch_types=[
            pltpu.VMEM((2, _CB, L), jnp.int32),
            pltpu.VMEM((rows_per_block, T), jnp.float32),
            pltpu.VMEM((rows_per_block, T), jnp.float32),
            pltpu.VMEM((b_per_w, T), jnp.float32),
            pltpu.SemaphoreType.DMA,
            pltpu.SemaphoreType.DMA,
            pltpu.SemaphoreType.DMA,
            pltpu.SemaphoreType.DMA,
        ],
    )
    def sc_embed_sum(x_hbm, tab_hbm, out_hbm, idx_v, rows0, rows1, out_v,
                     gsem0, gsem1, isem0, isem1):
        wid = lax.axis_index("s") * _NC + lax.axis_index("c")
        base = wid * b_per_w
        rows_bufs = (rows0, rows1)
        gsems = (gsem0, gsem1)
        isems = (isem0, isem1)

        def idx_src(kb):
            return x_hbm.at[pl.ds(base + kb * _CB, _CB)]

        def fire_gathers(kb_buf, rows_ref, sem):
            for i in range(_CB):
                off = 0
                for g in _SPLITS:
                    pltpu.async_copy(
                        tab_hbm.at[idx_v.at[kb_buf, i, pl.ds(off, g)]],
                        rows_ref.at[pl.ds(i * L + off, g)],
                        sem,
                    )
                    off += g

        def drain_gathers(rows_ref, sem):
            pltpu.make_async_copy(
                tab_hbm.at[pl.ds(0, rows_per_block)], rows_ref, sem
            ).wait()

        pltpu.sync_copy(idx_src(0), idx_v.at[0])
        fire_gathers(0, rows0, gsem0)
        pltpu.async_copy(idx_src(1), idx_v.at[1], isem1)

        def half_step(kb, cur):
            rows_c = rows_bufs[cur]
            rows_n = rows_bufs[1 - cur]

            @pl.when(kb + 1 < n_blocks)
            def _():
                pltpu.make_async_copy(
                    idx_src(kb + 1), idx_v.at[1 - cur], isems[1 - cur]
                ).wait()
                fire_gathers(1 - cur, rows_n, gsems[1 - cur])

            drain_gathers(rows_c, gsems[cur])

            @pl.when(kb + 2 < n_blocks)
            def _():
                pltpu.async_copy(idx_src(kb + 2), idx_v.at[cur], isems[cur])

            _sum_block(rows_c, out_v, kb * _CB, _CB, L)

        def body(t, carry):
            half_step(2 * t, 0)
            half_step(2 * t + 1, 1)
            return carry

        lax.fori_loop(0, n_blocks // 2, body, 0)
        pltpu.sync_copy(out_v, out_hbm.at[pl.ds(base, b_per_w)])

    return sc_embed_sum


def _logsoftmax_body(s_ref, b_ref, o_ref):
    s = s_ref[...] + b_ref[...]
    m = jnp.max(s, axis=-1, keepdims=True)
    e = jnp.exp(s - m)
    lse = jnp.log(jnp.sum(e, axis=-1, keepdims=True))
    o_ref[...] = (s - m) - lse


def _tc_log_softmax(scores, bias):
    B, T = scores.shape
    blk = min(2048, B)
    return pl.pallas_call(
        _logsoftmax_body,
        out_shape=jax.ShapeDtypeStruct((B, T), jnp.float32),
        grid=(B // blk,),
        in_specs=[
            pl.BlockSpec((blk, T), lambda i: (i, 0)),
            pl.BlockSpec((1, T), lambda i: (0, 0)),
        ],
        out_specs=pl.BlockSpec((blk, T), lambda i: (i, 0)),
    )(scores, bias.reshape(1, T))


def kernel(x, embed_weight, bow_bias):
    B, L = x.shape
    V, T = embed_weight.shape
    scores = _make_sc_embed_sum(B, V, T, L)(x, embed_weight)
    return _tc_log_softmax(scores, bow_bias)


# speedup vs baseline: 1.0528x; 1.0212x over previous
import functools

import jax
import jax.numpy as jnp
from jax import lax
from jax.experimental import pallas as pl
from jax.experimental.pallas import tpu as pltpu
from jax.experimental.pallas import tpu_sc as plsc


_NC = 2
_NS = 16
_NW = _NC * _NS
_LANES = 16

_CB = 8
_SPLITS = (104, 96)


def _sum_block(rows_ref, out_ref, out_row0, n_rows, l_per_row):
    unroll = 8
    steps = l_per_row // unroll

    for i in range(n_rows):
        flat0 = i * l_per_row

        def body(t, accs):
            accs = list(accs)
            r = flat0 + t * unroll
            for u in range(unroll):
                accs[2 * (u % 4)] = (
                    accs[2 * (u % 4)] + rows_ref[r + u, 0:16])
                accs[2 * (u % 4) + 1] = (
                    accs[2 * (u % 4) + 1] + rows_ref[r + u, 16:32])
            return tuple(accs)

        z = jnp.zeros((_LANES,), jnp.float32)
        accs = lax.fori_loop(0, steps, body, (z,) * 8)
        out_ref[out_row0 + i, 0:16] = (accs[0] + accs[2]) + (accs[4] + accs[6])
        out_ref[out_row0 + i, 16:32] = (accs[1] + accs[3]) + (accs[5] + accs[7])


def _make_sc_embed_sum(B, V, T, L):
    assert T == 32 and sum(_SPLITS) == L
    b_per_w = B // _NW
    n_blocks = b_per_w // _CB
    rows_per_block = _CB * L
    mesh = plsc.VectorSubcoreMesh(core_axis_name="c", subcore_axis_name="s")

    @functools.partial(
        pl.kernel,
        out_type=jax.ShapeDtypeStruct((B, T), jnp.float32),
        mesh=mesh,
        compiler_params=pltpu.CompilerParams(use_tc_tiling_on_sc=False),
        scratch_types=[
            pltpu.VMEM((2, _CB, L), jnp.int32),
            pltpu.VMEM((rows_per_block, T), jnp.float32),
            pltpu.VMEM((rows_per_block, T), jnp.float32),
            pltpu.VMEM((b_per_w, T), jnp.float32),
            pltpu.SemaphoreType.DMA,
            pltpu.SemaphoreType.DMA,
            pltpu.SemaphoreType.DMA,
            pltpu.SemaphoreType.DMA,
        ],
    )
    def sc_embed_sum(x_hbm, tab_hbm, out_hbm, idx_v, rows0, rows1, out_v,
                     gsem0, gsem1, isem0, isem1):
        wid = lax.axis_index("s") * _NC + lax.axis_index("c")
        base = wid * b_per_w
        rows_bufs = (rows0, rows1)
        gsems = (gsem0, gsem1)
        isems = (isem0, isem1)

        def idx_src(kb):
            return x_hbm.at[pl.ds(base + kb * _CB, _CB)]

        def fire_gathers(kb_buf, rows_ref, sem):
            for i in range(_CB):
                off = 0
                for g in _SPLITS:
                    pltpu.async_copy(
                        tab_hbm.at[idx_v.at[kb_buf, i, pl.ds(off, g)]],
                        rows_ref.at[pl.ds(i * L + off, g)],
                        sem,
                    )
                    off += g

        def drain_gathers_half(rows_ref, sem, h):
            half_rows = rows_per_block // 2
            pltpu.make_async_copy(
                tab_hbm.at[pl.ds(0, half_rows)],
                rows_ref.at[pl.ds(h * half_rows, half_rows)], sem
            ).wait()

        pltpu.sync_copy(idx_src(0), idx_v.at[0])
        fire_gathers(0, rows0, gsem0)
        pltpu.async_copy(idx_src(1), idx_v.at[1], isem1)

        def half_step(kb, cur):
            rows_c = rows_bufs[cur]
            rows_n = rows_bufs[1 - cur]

            @pl.when(kb + 1 < n_blocks)
            def _():
                pltpu.make_async_copy(
                    idx_src(kb + 1), idx_v.at[1 - cur], isems[1 - cur]
                ).wait()
                fire_gathers(1 - cur, rows_n, gsems[1 - cur])

            drain_gathers_half(rows_c, gsems[cur], 0)

            @pl.when(kb + 2 < n_blocks)
            def _():
                pltpu.async_copy(idx_src(kb + 2), idx_v.at[cur], isems[cur])

            _sum_block(rows_c, out_v, kb * _CB, _CB // 2, L)
            drain_gathers_half(rows_c, gsems[cur], 1)
            _sum_block(rows_c.at[pl.ds(rows_per_block // 2, rows_per_block // 2)],
                       out_v, kb * _CB + _CB // 2, _CB // 2, L)

        def body(t, carry):
            half_step(2 * t, 0)
            half_step(2 * t + 1, 1)
            return carry

        lax.fori_loop(0, n_blocks // 2, body, 0)
        pltpu.sync_copy(out_v, out_hbm.at[pl.ds(base, b_per_w)])

    return sc_embed_sum


def _logsoftmax_body(s_ref, b_ref, o_ref):
    s = s_ref[...] + b_ref[...]
    m = jnp.max(s, axis=-1, keepdims=True)
    e = jnp.exp(s - m)
    lse = jnp.log(jnp.sum(e, axis=-1, keepdims=True))
    o_ref[...] = (s - m) - lse


def _tc_log_softmax(scores, bias):
    B, T = scores.shape
    blk = min(2048, B)
    return pl.pallas_call(
        _logsoftmax_body,
        out_shape=jax.ShapeDtypeStruct((B, T), jnp.float32),
        grid=(B // blk,),
        in_specs=[
            pl.BlockSpec((blk, T), lambda i: (i, 0)),
            pl.BlockSpec((1, T), lambda i: (0, 0)),
        ],
        out_specs=pl.BlockSpec((blk, T), lambda i: (i, 0)),
    )(scores, bias.reshape(1, T))


def kernel(x, embed_weight, bow_bias):
    B, L = x.shape
    V, T = embed_weight.shape
    scores = _make_sc_embed_sum(B, V, T, L)(x, embed_weight)
    return _tc_log_softmax(scores, bow_bias)


# speedup vs baseline: 1.0540x; 1.0012x over previous
import functools

import jax
import jax.numpy as jnp
from jax import lax
from jax.experimental import pallas as pl
from jax.experimental.pallas import tpu as pltpu
from jax.experimental.pallas import tpu_sc as plsc


_NC = 2
_NS = 16
_NW = _NC * _NS
_LANES = 16

_CB = 8
_SPLITS = (104, 96)


def _sum_block(rows_ref, out_ref, out_row0, n_rows, l_per_row):
    unroll = 8
    steps = l_per_row // unroll

    for i in range(n_rows):
        flat0 = i * l_per_row

        def body(t, accs):
            accs = list(accs)
            r = flat0 + t * unroll
            for u in range(unroll):
                accs[2 * (u % 4)] = (
                    accs[2 * (u % 4)] + rows_ref[r + u, 0:16])
                accs[2 * (u % 4) + 1] = (
                    accs[2 * (u % 4) + 1] + rows_ref[r + u, 16:32])
            return tuple(accs)

        z = jnp.zeros((_LANES,), jnp.float32)
        accs = lax.fori_loop(0, steps, body, (z,) * 8)
        out_ref[out_row0 + i, 0:16] = (accs[0] + accs[2]) + (accs[4] + accs[6])
        out_ref[out_row0 + i, 16:32] = (accs[1] + accs[3]) + (accs[5] + accs[7])


def _make_sc_embed_sum(B, V, T, L):
    assert T == 32 and sum(_SPLITS) == L
    b_per_w = B // _NW
    n_blocks = b_per_w // _CB
    rows_per_block = _CB * L
    mesh = plsc.VectorSubcoreMesh(core_axis_name="c", subcore_axis_name="s")

    @functools.partial(
        pl.kernel,
        out_type=jax.ShapeDtypeStruct((B, T), jnp.float32),
        mesh=mesh,
        compiler_params=pltpu.CompilerParams(use_tc_tiling_on_sc=False),
        scratch_types=[
            pltpu.VMEM((2, _CB, L), jnp.int32),
            pltpu.VMEM((rows_per_block, T), jnp.float32),
            pltpu.VMEM((rows_per_block, T), jnp.float32),
            pltpu.VMEM((b_per_w, T), jnp.float32),
            pltpu.SemaphoreType.DMA,
            pltpu.SemaphoreType.DMA,
            pltpu.SemaphoreType.DMA,
            pltpu.SemaphoreType.DMA,
        ],
    )
    def sc_embed_sum(x_hbm, tab_hbm, out_hbm, idx_v, rows0, rows1, out_v,
                     gsem0, gsem1, isem0, isem1):
        wid = lax.axis_index("s") * _NC + lax.axis_index("c")
        base = wid * b_per_w
        rows_bufs = (rows0, rows1)
        gsems = (gsem0, gsem1)
        isems = (isem0, isem1)

        def idx_src(kb):
            return x_hbm.at[pl.ds(base + kb * _CB, _CB)]

        def fire_gathers(kb_buf, rows_ref, sem):
            for i in range(_CB):
                off = 0
                for g in _SPLITS:
                    pltpu.async_copy(
                        tab_hbm.at[idx_v.at[kb_buf, i, pl.ds(off, g)]],
                        rows_ref.at[pl.ds(i * L + off, g)],
                        sem,
                    )
                    off += g

        def drain_gathers_part(rows_ref, sem, h, parts):
            part_rows = rows_per_block // parts
            pltpu.make_async_copy(
                tab_hbm.at[pl.ds(0, part_rows)],
                rows_ref.at[pl.ds(h * part_rows, part_rows)], sem
            ).wait()

        pltpu.sync_copy(idx_src(0), idx_v.at[0])
        fire_gathers(0, rows0, gsem0)
        pltpu.async_copy(idx_src(1), idx_v.at[1], isem1)

        def half_step(kb, cur):
            rows_c = rows_bufs[cur]
            rows_n = rows_bufs[1 - cur]

            @pl.when(kb + 1 < n_blocks)
            def _():
                pltpu.make_async_copy(
                    idx_src(kb + 1), idx_v.at[1 - cur], isems[1 - cur]
                ).wait()
                fire_gathers(1 - cur, rows_n, gsems[1 - cur])

            parts = 4
            rows_pp = _CB // parts
            drain_gathers_part(rows_c, gsems[cur], 0, parts)

            @pl.when(kb + 2 < n_blocks)
            def _():
                pltpu.async_copy(idx_src(kb + 2), idx_v.at[cur], isems[cur])

            _sum_block(rows_c, out_v, kb * _CB, rows_pp, L)
            for h in range(1, parts):
                drain_gathers_part(rows_c, gsems[cur], h, parts)
                _sum_block(
                    rows_c.at[pl.ds(h * (rows_per_block // parts),
                                    rows_per_block // parts)],
                    out_v, kb * _CB + h * rows_pp, rows_pp, L)

        def body(t, carry):
            half_step(2 * t, 0)
            half_step(2 * t + 1, 1)
            return carry

        lax.fori_loop(0, n_blocks // 2, body, 0)
        pltpu.sync_copy(out_v, out_hbm.at[pl.ds(base, b_per_w)])

    return sc_embed_sum


def _logsoftmax_body(s_ref, b_ref, o_ref):
    s = s_ref[...] + b_ref[...]
    m = jnp.max(s, axis=-1, keepdims=True)
    e = jnp.exp(s - m)
    lse = jnp.log(jnp.sum(e, axis=-1, keepdims=True))
    o_ref[...] = (s - m) - lse


def _tc_log_softmax(scores, bias):
    B, T = scores.shape
    blk = min(2048, B)
    return pl.pallas_call(
        _logsoftmax_body,
        out_shape=jax.ShapeDtypeStruct((B, T), jnp.float32),
        grid=(B // blk,),
        in_specs=[
            pl.BlockSpec((blk, T), lambda i: (i, 0)),
            pl.BlockSpec((1, T), lambda i: (0, 0)),
        ],
        out_specs=pl.BlockSpec((blk, T), lambda i: (i, 0)),
    )(scores, bias.reshape(1, T))


def kernel(x, embed_weight, bow_bias):
    B, L = x.shape
    V, T = embed_weight.shape
    scores = _make_sc_embed_sum(B, V, T, L)(x, embed_weight)
    return _tc_log_softmax(scores, bow_bias)
